# baseline (speedup 1.0000x reference)
import functools

import jax
import jax.numpy as jnp
from jax import lax
from jax.experimental import pallas as pl
from jax.experimental.pallas import tpu as pltpu
from jax.experimental.pallas import tpu_sc as plsc

N = 10000
E = 160000
DI = 256
DH = 512
DO = 40
NC = 2
NS = 16
NW = NC * NS
CHUNK = 128
NCH = E // CHUNK
HALF = DI // 2
BK = 1000

_sc_mesh = plsc.VectorSubcoreMesh(core_axis_name="c", subcore_axis_name="s")
_sc_params = pltpu.CompilerParams(use_tc_tiling_on_sc=False)


def _vcopy(src_ref, dst_ref, n):
    offs = list(range(0, n - 15, 16))
    if offs[-1] != n - 16:
        offs.append(n - 16)
    for o in offs:
        dst_ref[pl.ds(o, 16)] = src_ref[pl.ds(o, 16)]


CH = 128
NROW = 1280
ND = NROW // NW
CH_D = CH


@functools.partial(
    pl.kernel,
    out_type=jax.ShapeDtypeStruct((NC * N, 8), jnp.float32),
    mesh=_sc_mesh,
    compiler_params=_sc_params,
    scratch_types=[
        pltpu.VMEM((CH_D,), jnp.int32),
        pltpu.VMEM((CH_D,), jnp.int32),
        pltpu.VMEM((CH_D,), jnp.int32),
        pltpu.VMEM((CH_D,), jnp.int32),
        pltpu.VMEM((CH_D, 8), jnp.float32),
        pltpu.VMEM((1000, 8), jnp.float32),
        pltpu.SemaphoreType.DMA,
        pltpu.SemaphoreType.DMA,
        pltpu.SemaphoreType.DMA,
        pltpu.SemaphoreType.DMA,
        pltpu.VMEM_SHARED((N + 128, 8), jnp.float32),
    ],
)
def _deg_kernel(dst_hbm, ones_hbm, zeros_hbm, out_hbm,
                dst0, dst1, sdst0, sdst1, onesv, zbuf,
                isem0, isem1, ssem0, ssem1, acc):
    c = lax.axis_index("c")
    s = lax.axis_index("s")
    w = s * NC + c
    dstb = (dst0, dst1)
    sdstb = (sdst0, sdst1)
    isems = (isem0, isem1)
    ssems = (ssem0, ssem1)
    pltpu.sync_copy(zeros_hbm, zbuf)
    pltpu.sync_copy(ones_hbm, onesv)

    @pl.when(s < 10)
    def _():
        pltpu.sync_copy(zbuf, acc.at[pl.ds(s * 1000, 1000)])

    plsc.subcore_barrier()

    def off_of(k):
        return (w + k * NW) * CH_D

    pltpu.async_copy(dst_hbm.at[pl.ds(off_of(0), CH_D)], dst0, isem0)

    def iter_k(k, b):
        nb = 1 - b
        pltpu.make_async_copy(dst_hbm.at[pl.ds(0, CH_D)], dstb[b],
                              isems[b]).wait()

        @pl.when(k >= 2)
        def _():
            pltpu.make_async_copy(onesv, acc.at[sdstb[b]], ssems[b]).wait()

        _vcopy(dstb[b], sdstb[b], CH_D)

        @pl.when(k + 1 < ND)
        def _():
            pltpu.async_copy(dst_hbm.at[pl.ds(off_of(k + 1), CH_D)],
                             dstb[nb], isems[nb])

        pltpu.async_copy(onesv, acc.at[sdstb[b]], ssems[b], add=True)

    def pair(k2, carry):
        iter_k(2 * k2, 0)
        iter_k(2 * k2 + 1, 1)
        return carry

    lax.fori_loop(0, ND // 2, pair, 0)
    if ND % 2:
        iter_k(ND - 1, (ND - 1) % 2)
    pltpu.make_async_copy(onesv, acc.at[sdstb[(ND - 2) % 2]],
                          ssems[(ND - 2) % 2]).wait()
    pltpu.make_async_copy(onesv, acc.at[sdstb[(ND - 1) % 2]],
                          ssems[(ND - 1) % 2]).wait()
    plsc.subcore_barrier()

    @pl.when(s < 10)
    def _():
        pltpu.sync_copy(acc.at[pl.ds(s * 1000, 1000)], zbuf)
        pltpu.sync_copy(zbuf, out_hbm.at[pl.ds(c * N + s * 1000, 1000)])


CH_1 = 80
N1 = (NROW * CH) // (CH_1 * NS)


@functools.partial(
    pl.kernel,
    out_type=jax.ShapeDtypeStruct((NC * N, HALF), jnp.float32),
    mesh=_sc_mesh,
    compiler_params=_sc_params,
    scratch_types=[
        pltpu.VMEM((CH_1,), jnp.int32),
        pltpu.VMEM((CH_1,), jnp.int32),
        pltpu.VMEM((CH_1,), jnp.int32),
        pltpu.VMEM((CH_1,), jnp.int32),
        pltpu.VMEM((CH_1,), jnp.int32),
        pltpu.VMEM((CH_1,), jnp.int32),
        pltpu.VMEM((CH_1,), jnp.int32),
        pltpu.VMEM((CH_1,), jnp.int32),
        pltpu.VMEM((CH_1, HALF), jnp.float32),
        pltpu.VMEM((CH_1, HALF), jnp.float32),
        pltpu.VMEM((125, HALF), jnp.float32),
        pltpu.SemaphoreType.DMA,
        pltpu.SemaphoreType.DMA,
        pltpu.SemaphoreType.DMA,
        pltpu.SemaphoreType.DMA,
        pltpu.SemaphoreType.DMA,
        pltpu.SemaphoreType.DMA,
        pltpu.VMEM_SHARED((N + 128, HALF), jnp.float32),
    ],
)
def _s1_kernel(xs_hbm, src_hbm, dst_hbm, zeros_hbm, out_hbm,
               src0, src1, dst0, dst1, idx0, idx1, sdst0, sdst1,
               rows0, rows1, zbuf,
               isem0, isem1, gsem0, gsem1, ssem0, ssem1, acc):
    c = lax.axis_index("c")
    s = lax.axis_index("s")
    stripe = N // NS
    pltpu.sync_copy(zeros_hbm, zbuf)
    for r in range(stripe // 125):
        pltpu.sync_copy(zbuf, acc.at[pl.ds(s * stripe + r * 125, 125)])
    plsc.subcore_barrier()
    base = c * N
    srcb = (src0, src1)
    dstb = (dst0, dst1)
    idxb = (idx0, idx1)
    rowsb = (rows0, rows1)
    sdstb = (sdst0, sdst1)
    isems = (isem0, isem1)
    gsems = (gsem0, gsem1)
    ssems = (ssem0, ssem1)

    def off_of(k):
        return (s + k * NS) * CH_1

    def load_idx(k, b):
        pltpu.async_copy(src_hbm.at[pl.ds(off_of(k), CH_1)], srcb[b], isems[b])
        pltpu.async_copy(dst_hbm.at[pl.ds(off_of(k), CH_1)], dstb[b], isems[b])

    def wait_idx(b):
        pltpu.make_async_copy(src_hbm.at[pl.ds(0, CH_1)], srcb[b],
                              isems[b]).wait()
        pltpu.make_async_copy(dst_hbm.at[pl.ds(0, CH_1)], dstb[b],
                              isems[b]).wait()

    def prep_gather(k, b):
        offs = list(range(0, CH_1 - 15, 16))
        if offs[-1] != CH_1 - 16:
            offs.append(CH_1 - 16)
        for o in offs:
            idxb[b][pl.ds(o, 16)] = srcb[b][pl.ds(o, 16)] + base
        pltpu.async_copy(xs_hbm.at[idxb[b]], rowsb[b], gsems[b])

    load_idx(0, 0)
    wait_idx(0)
    prep_gather(0, 0)
    load_idx(1, 1)

    def iter_k(k, b):
        nb = 1 - b
        pltpu.make_async_copy(xs_hbm.at[idxb[b]], rowsb[b], gsems[b]).wait()
        _vcopy(dstb[b], sdstb[b], CH_1)

        @pl.when(k + 1 < N1)
        def _():
            wait_idx(nb)

            @pl.when(k >= 1)
            def _():
                pltpu.make_async_copy(rowsb[nb], acc.at[sdstb[nb]],
                                      ssems[nb]).wait()

            prep_gather(k + 1, nb)

        pltpu.async_copy(rowsb[b], acc.at[sdstb[b]], ssems[b], add=True)

        @pl.when(k + 2 < N1)
        def _():
            load_idx(k + 2, b)

    def pair(k2, carry):
        iter_k(2 * k2, 0)
        iter_k(2 * k2 + 1, 1)
        return carry

    lax.fori_loop(0, N1 // 2, pair, 0)
    if N1 % 2:
        iter_k(N1 - 1, (N1 - 1) % 2)
    pltpu.make_async_copy(rowsb[(N1 - 2) % 2], acc.at[sdstb[(N1 - 2) % 2]],
                          ssems[(N1 - 2) % 2]).wait()
    pltpu.make_async_copy(rowsb[(N1 - 1) % 2], acc.at[sdstb[(N1 - 1) % 2]],
                          ssems[(N1 - 1) % 2]).wait()
    plsc.subcore_barrier()
    for r in range(stripe // 125):
        pltpu.sync_copy(acc.at[pl.ds(s * stripe + r * 125, 125)], zbuf)
        pltpu.sync_copy(zbuf, out_hbm.at[pl.ds(c * N + s * stripe + r * 125, 125)])


@functools.partial(
    pl.kernel,
    out_type=jax.ShapeDtypeStruct((NC * N, DO), jnp.float32),
    mesh=_sc_mesh,
    compiler_params=_sc_params,
    scratch_types=[
        pltpu.VMEM((CH_D,), jnp.int32),
        pltpu.VMEM((CH_D,), jnp.int32),
        pltpu.VMEM((CH_D,), jnp.int32),
        pltpu.VMEM((CH_D,), jnp.int32),
        pltpu.VMEM((CH_D,), jnp.int32),
        pltpu.VMEM((CH_D,), jnp.int32),
        pltpu.VMEM((CH_D,), jnp.int32),
        pltpu.VMEM((CH_D,), jnp.int32),
        pltpu.VMEM((CH_D, DO), jnp.float32),
        pltpu.VMEM((CH_D, DO), jnp.float32),
        pltpu.VMEM((1000, DO), jnp.float32),
        pltpu.SemaphoreType.DMA,
        pltpu.SemaphoreType.DMA,
        pltpu.SemaphoreType.DMA,
        pltpu.SemaphoreType.DMA,
        pltpu.SemaphoreType.DMA,
        pltpu.SemaphoreType.DMA,
        pltpu.VMEM_SHARED((N + 128, DO), jnp.float32),
    ],
)
def _s2_kernel(ts_hbm, src_hbm, dst_hbm, zeros_hbm, out_hbm,
               src0, src1, dst0, dst1, sidx0, sidx1, sdst0, sdst1,
               rows0, rows1, zbuf,
               isem0, isem1, gsem0, gsem1, ssem0, ssem1, acc):
    c = lax.axis_index("c")
    s = lax.axis_index("s")
    w = s * NC + c
    srcb = (src0, src1)
    dstb = (dst0, dst1)
    sidxb = (sidx0, sidx1)
    sdstb = (sdst0, sdst1)
    rowsb = (rows0, rows1)
    isems = (isem0, isem1)
    gsems = (gsem0, gsem1)
    ssems = (ssem0, ssem1)
    pltpu.sync_copy(zeros_hbm, zbuf)

    @pl.when(s < 10)
    def _():
        pltpu.sync_copy(zbuf, acc.at[pl.ds(s * 1000, 1000)])

    plsc.subcore_barrier()

    def off_of(k):
        return (w + k * NW) * CH_D

    def load_idx(k, b):
        pltpu.async_copy(src_hbm.at[pl.ds(off_of(k), CH_D)], srcb[b], isems[b])
        pltpu.async_copy(dst_hbm.at[pl.ds(off_of(k), CH_D)], dstb[b], isems[b])

    def wait_idx(b):
        pltpu.make_async_copy(src_hbm.at[pl.ds(0, CH_D)], srcb[b],
                              isems[b]).wait()
        pltpu.make_async_copy(dst_hbm.at[pl.ds(0, CH_D)], dstb[b],
                              isems[b]).wait()

    load_idx(0, 0)
    wait_idx(0)
    _vcopy(src0, sidx0, CH_D)
    pltpu.async_copy(ts_hbm.at[sidx0], rows0, gsem0)
    load_idx(1, 1)

    def iter_k(k, b):
        nb = 1 - b
        pltpu.make_async_copy(ts_hbm.at[sidxb[b]], rowsb[b], gsems[b]).wait()
        _vcopy(dstb[b], sdstb[b], CH_D)

        @pl.when(k + 1 < ND)
        def _():
            wait_idx(nb)

            @pl.when(k >= 1)
            def _():
                pltpu.make_async_copy(rowsb[nb], acc.at[sdstb[nb]],
                                      ssems[nb]).wait()

            _vcopy(srcb[nb], sidxb[nb], CH_D)
            pltpu.async_copy(ts_hbm.at[sidxb[nb]], rowsb[nb], gsems[nb])

        pltpu.async_copy(rowsb[b], acc.at[sdstb[b]], ssems[b], add=True)

        @pl.when(k + 2 < ND)
        def _():
            load_idx(k + 2, b)

    def pair(k2, carry):
        iter_k(2 * k2, 0)
        iter_k(2 * k2 + 1, 1)
        return carry

    lax.fori_loop(0, ND // 2, pair, 0)
    if ND % 2:
        iter_k(ND - 1, (ND - 1) % 2)
    pltpu.make_async_copy(rowsb[(ND - 2) % 2], acc.at[sdstb[(ND - 2) % 2]],
                          ssems[(ND - 2) % 2]).wait()
    pltpu.make_async_copy(rowsb[(ND - 1) % 2], acc.at[sdstb[(ND - 1) % 2]],
                          ssems[(ND - 1) % 2]).wait()
    plsc.subcore_barrier()

    @pl.when(s < 10)
    def _():
        pltpu.sync_copy(acc.at[pl.ds(s * 1000, 1000)], zbuf)
        pltpu.sync_copy(zbuf, out_hbm.at[pl.ds(c * N + s * 1000, 1000)])


def _scale_body(x_ref, degp_ref, xs_ref, dinv_ref):
    deg = degp_ref[0, :, :1] + degp_ref[1, :, :1] + 1.0
    dinv = lax.rsqrt(deg)
    dinv_ref[...] = dinv
    xb = x_ref[...] * dinv
    xs_ref[0] = xb[:, :HALF]
    xs_ref[1] = xb[:, HALF:]


def _tc_scale(x, degp):
    return pl.pallas_call(
        _scale_body,
        grid=(N // BK,),
        in_specs=[
            pl.BlockSpec((BK, DI), lambda i: (i, 0)),
            pl.BlockSpec((2, BK, 8), lambda i: (0, i, 0)),
        ],
        out_specs=[
            pl.BlockSpec((2, BK, HALF), lambda i: (0, i, 0)),
            pl.BlockSpec((BK, 1), lambda i: (i, 0)),
        ],
        out_shape=[
            jax.ShapeDtypeStruct((2, N, HALF), jnp.float32),
            jax.ShapeDtypeStruct((N, 1), jnp.float32),
        ],
    )(x, degp)


def _mm_body(s1a, s1b, xsa, xsb, dinv, w1a, w1b, b1r, w2, ts_ref):
    dv = dinv[...]
    a = (s1a[...] + xsa[...]) * dv
    b = (s1b[...] + xsb[...]) * dv
    h = jnp.dot(a, w1a[...], preferred_element_type=jnp.float32)
    h = h + jnp.dot(b, w1b[...], preferred_element_type=jnp.float32)
    h = jnp.maximum(h + b1r[...], 0.0)
    t = jnp.dot(h, w2[...], preferred_element_type=jnp.float32)
    ts_ref[...] = t * dv


def _tc_mm(s1a, s1b, xsa, xsb, dinv, W1a, W1b, b1r, W2):
    full = lambda shape: pl.BlockSpec(shape, lambda i: tuple(0 for _ in shape))
    return pl.pallas_call(
        _mm_body,
        grid=(N // BK,),
        in_specs=[
            pl.BlockSpec((BK, HALF), lambda i: (i, 0)),
            pl.BlockSpec((BK, HALF), lambda i: (i, 0)),
            pl.BlockSpec((BK, HALF), lambda i: (i, 0)),
            pl.BlockSpec((BK, HALF), lambda i: (i, 0)),
            pl.BlockSpec((BK, 1), lambda i: (i, 0)),
            full((HALF, DH)),
            full((HALF, DH)),
            full((1, DH)),
            full((DH, DO)),
        ],
        out_specs=pl.BlockSpec((BK, DO), lambda i: (i, 0)),
        out_shape=jax.ShapeDtypeStruct((N, DO), jnp.float32),
    )(s1a, s1b, xsa, xsb, dinv, W1a, W1b, b1r, W2)


def _fin_body(s2_ref, ts_ref, dinv, b2r, out_ref):
    acc = s2_ref[0] + s2_ref[1] + ts_ref[...]
    out_ref[...] = acc * dinv[...] + b2r[...]


def _tc_fin(s2, ts, dinv, b2r):
    return pl.pallas_call(
        _fin_body,
        grid=(N // BK,),
        in_specs=[
            pl.BlockSpec((2, BK, DO), lambda i: (0, i, 0)),
            pl.BlockSpec((BK, DO), lambda i: (i, 0)),
            pl.BlockSpec((BK, 1), lambda i: (i, 0)),
            pl.BlockSpec((1, DO), lambda i: (0, 0)),
        ],
        out_specs=pl.BlockSpec((BK, DO), lambda i: (i, 0)),
        out_shape=jax.ShapeDtypeStruct((N, DO), jnp.float32),
    )(s2, ts, dinv, b2r)


def kernel(x, edge_index, W1, b1, W2, b2):
    x = x.astype(jnp.float32)
    pad = NROW * CH - E
    src = jnp.concatenate(
        [edge_index[0].astype(jnp.int32), jnp.zeros((pad,), jnp.int32)])
    dst = jnp.concatenate(
        [edge_index[1].astype(jnp.int32),
         N + (jnp.arange(pad, dtype=jnp.int32) % 128)])

    ones_c = jnp.ones((CH_D, 8), jnp.float32)
    zeros_1 = jnp.zeros((1000, 8), jnp.float32)
    zeros_h = jnp.zeros((125, HALF), jnp.float32)
    zeros_o = jnp.zeros((1000, DO), jnp.float32)

    degp = _deg_kernel(dst, ones_c, zeros_1)
    xs, dinv = _tc_scale(x, degp.reshape(NC, N, 8))

    s1 = _s1_kernel(xs.reshape(NC * N, HALF), src, dst, zeros_h)
    s1 = s1.reshape(NC, N, HALF)

    ts = _tc_mm(s1[0], s1[1], xs[0], xs[1], dinv,
                W1[:HALF], W1[HALF:], b1.reshape(1, DH), W2)

    s2 = _s2_kernel(ts, src, dst, zeros_o)
    return _tc_fin(s2.reshape(NC, N, DO), ts, dinv, b2.reshape(1, DO))


# speedup vs baseline: 1.3402x; 1.3402x over previous
import functools

import jax
import jax.numpy as jnp
from jax import lax
from jax.experimental import pallas as pl
from jax.experimental.pallas import tpu as pltpu
from jax.experimental.pallas import tpu_sc as plsc

N = 10000
E = 160000
DI = 256
DH = 512
DO = 40
NC = 2
NS = 16
NW = NC * NS
CHUNK = 128
NCH = E // CHUNK
HALF = DI // 2
BK = 1000

_sc_mesh = plsc.VectorSubcoreMesh(core_axis_name="c", subcore_axis_name="s")
_sc_params = pltpu.CompilerParams(use_tc_tiling_on_sc=False)


def _vcopy(src_ref, dst_ref, n):
    offs = list(range(0, n - 15, 16))
    if offs[-1] != n - 16:
        offs.append(n - 16)
    for o in offs:
        dst_ref[pl.ds(o, 16)] = src_ref[pl.ds(o, 16)]


CH = 128
NROW = 1280
ND = NROW // NW
CH_D = CH


@functools.partial(
    pl.kernel,
    out_type=jax.ShapeDtypeStruct((NC * N, 8), jnp.float32),
    mesh=_sc_mesh,
    compiler_params=_sc_params,
    scratch_types=[
        pltpu.VMEM((CH_D,), jnp.int32),
        pltpu.VMEM((CH_D,), jnp.int32),
        pltpu.VMEM((CH_D,), jnp.int32),
        pltpu.VMEM((CH_D,), jnp.int32),
        pltpu.VMEM((CH_D, 8), jnp.float32),
        pltpu.VMEM((1000, 8), jnp.float32),
        pltpu.SemaphoreType.DMA,
        pltpu.SemaphoreType.DMA,
        pltpu.SemaphoreType.DMA,
        pltpu.SemaphoreType.DMA,
        pltpu.VMEM_SHARED((N + 128, 8), jnp.float32),
    ],
)
def _deg_kernel(dst_hbm, ones_hbm, zeros_hbm, out_hbm,
                dst0, dst1, sdst0, sdst1, onesv, zbuf,
                isem0, isem1, ssem0, ssem1, acc):
    c = lax.axis_index("c")
    s = lax.axis_index("s")
    w = s * NC + c
    dstb = (dst0, dst1)
    sdstb = (sdst0, sdst1)
    isems = (isem0, isem1)
    ssems = (ssem0, ssem1)
    pltpu.sync_copy(zeros_hbm, zbuf)
    pltpu.sync_copy(ones_hbm, onesv)

    @pl.when(s < 10)
    def _():
        pltpu.sync_copy(zbuf, acc.at[pl.ds(s * 1000, 1000)])

    plsc.subcore_barrier()

    def off_of(k):
        return (w + k * NW) * CH_D

    pltpu.async_copy(dst_hbm.at[pl.ds(off_of(0), CH_D)], dst0, isem0)

    def iter_k(k, b):
        nb = 1 - b
        pltpu.make_async_copy(dst_hbm.at[pl.ds(0, CH_D)], dstb[b],
                              isems[b]).wait()

        @pl.when(k >= 2)
        def _():
            pltpu.make_async_copy(onesv, acc.at[sdstb[b]], ssems[b]).wait()

        _vcopy(dstb[b], sdstb[b], CH_D)

        @pl.when(k + 1 < ND)
        def _():
            pltpu.async_copy(dst_hbm.at[pl.ds(off_of(k + 1), CH_D)],
                             dstb[nb], isems[nb])

        pltpu.async_copy(onesv, acc.at[sdstb[b]], ssems[b], add=True)

    def pair(k2, carry):
        iter_k(2 * k2, 0)
        iter_k(2 * k2 + 1, 1)
        return carry

    lax.fori_loop(0, ND // 2, pair, 0)
    if ND % 2:
        iter_k(ND - 1, (ND - 1) % 2)
    pltpu.make_async_copy(onesv, acc.at[sdstb[(ND - 2) % 2]],
                          ssems[(ND - 2) % 2]).wait()
    pltpu.make_async_copy(onesv, acc.at[sdstb[(ND - 1) % 2]],
                          ssems[(ND - 1) % 2]).wait()
    plsc.subcore_barrier()

    @pl.when(s < 10)
    def _():
        pltpu.sync_copy(acc.at[pl.ds(s * 1000, 1000)], zbuf)
        pltpu.sync_copy(zbuf, out_hbm.at[pl.ds(c * N + s * 1000, 1000)])


CH_1 = 80
N1 = E // (CH_1 * NS)


@functools.partial(
    pl.kernel,
    out_type=jax.ShapeDtypeStruct((NC * N, HALF), jnp.float32),
    mesh=_sc_mesh,
    compiler_params=_sc_params,
    scratch_types=[
        pltpu.VMEM((CH_1,), jnp.int32),
        pltpu.VMEM((CH_1,), jnp.int32),
        pltpu.VMEM((CH_1,), jnp.int32),
        pltpu.VMEM((CH_1,), jnp.int32),
        pltpu.VMEM((CH_1,), jnp.int32),
        pltpu.VMEM((CH_1,), jnp.int32),
        pltpu.VMEM((CH_1,), jnp.int32),
        pltpu.VMEM((CH_1,), jnp.int32),
        pltpu.VMEM((CH_1, HALF), jnp.float32),
        pltpu.VMEM((CH_1, HALF), jnp.float32),
        pltpu.VMEM((125, HALF), jnp.float32),
        pltpu.SemaphoreType.DMA,
        pltpu.SemaphoreType.DMA,
        pltpu.SemaphoreType.DMA,
        pltpu.SemaphoreType.DMA,
        pltpu.SemaphoreType.DMA,
        pltpu.SemaphoreType.DMA,
        pltpu.VMEM_SHARED((N + 128, HALF), jnp.float32),
    ],
)
def _s1_kernel(xs_hbm, src_hbm, dst_hbm, zeros_hbm, out_hbm,
               src0, src1, dst0, dst1, idx0, idx1, sdst0, sdst1,
               rows0, rows1, zbuf,
               isem0, isem1, gsem0, gsem1, ssem0, ssem1, acc):
    c = lax.axis_index("c")
    s = lax.axis_index("s")
    stripe = N // NS
    pltpu.sync_copy(zeros_hbm, zbuf)
    for r in range(stripe // 125):
        pltpu.sync_copy(zbuf, acc.at[pl.ds(s * stripe + r * 125, 125)])
    plsc.subcore_barrier()
    base = c * N
    srcb = (src0, src1)
    dstb = (dst0, dst1)
    idxb = (idx0, idx1)
    rowsb = (rows0, rows1)
    sdstb = (sdst0, sdst1)
    isems = (isem0, isem1)
    gsems = (gsem0, gsem1)
    ssems = (ssem0, ssem1)

    def off_of(k):
        return (s + k * NS) * CH_1

    def load_idx(k, b):
        pltpu.async_copy(src_hbm.at[pl.ds(off_of(k), CH_1)], srcb[b], isems[b])
        pltpu.async_copy(dst_hbm.at[pl.ds(off_of(k), CH_1)], dstb[b], isems[b])

    def wait_idx(b):
        pltpu.make_async_copy(src_hbm.at[pl.ds(0, CH_1)], srcb[b],
                              isems[b]).wait()
        pltpu.make_async_copy(dst_hbm.at[pl.ds(0, CH_1)], dstb[b],
                              isems[b]).wait()

    def prep_gather(k, b):
        offs = list(range(0, CH_1 - 15, 16))
        if offs[-1] != CH_1 - 16:
            offs.append(CH_1 - 16)
        for o in offs:
            idxb[b][pl.ds(o, 16)] = srcb[b][pl.ds(o, 16)] + base
        pltpu.async_copy(xs_hbm.at[idxb[b]], rowsb[b], gsems[b])

    load_idx(0, 0)
    wait_idx(0)
    prep_gather(0, 0)
    load_idx(1, 1)

    def iter_k(k, b):
        nb = 1 - b
        pltpu.make_async_copy(xs_hbm.at[idxb[b]], rowsb[b], gsems[b]).wait()
        _vcopy(dstb[b], sdstb[b], CH_1)

        @pl.when(k + 1 < N1)
        def _():
            wait_idx(nb)

            @pl.when(k >= 1)
            def _():
                pltpu.make_async_copy(rowsb[nb], acc.at[sdstb[nb]],
                                      ssems[nb]).wait()

            prep_gather(k + 1, nb)

        pltpu.async_copy(rowsb[b], acc.at[sdstb[b]], ssems[b], add=True)

        @pl.when(k + 2 < N1)
        def _():
            load_idx(k + 2, b)

    def pair(k2, carry):
        iter_k(2 * k2, 0)
        iter_k(2 * k2 + 1, 1)
        return carry

    lax.fori_loop(0, N1 // 2, pair, 0)
    if N1 % 2:
        iter_k(N1 - 1, (N1 - 1) % 2)
    pltpu.make_async_copy(rowsb[(N1 - 2) % 2], acc.at[sdstb[(N1 - 2) % 2]],
                          ssems[(N1 - 2) % 2]).wait()
    pltpu.make_async_copy(rowsb[(N1 - 1) % 2], acc.at[sdstb[(N1 - 1) % 2]],
                          ssems[(N1 - 1) % 2]).wait()
    plsc.subcore_barrier()
    for r in range(stripe // 125):
        pltpu.sync_copy(acc.at[pl.ds(s * stripe + r * 125, 125)], zbuf)
        pltpu.sync_copy(zbuf, out_hbm.at[pl.ds(c * N + s * stripe + r * 125, 125)])


@functools.partial(
    pl.kernel,
    out_type=jax.ShapeDtypeStruct((NC * N, DO), jnp.float32),
    mesh=_sc_mesh,
    compiler_params=_sc_params,
    scratch_types=[
        pltpu.VMEM((CH_D,), jnp.int32),
        pltpu.VMEM((CH_D,), jnp.int32),
        pltpu.VMEM((CH_D,), jnp.int32),
        pltpu.VMEM((CH_D,), jnp.int32),
        pltpu.VMEM((CH_D,), jnp.int32),
        pltpu.VMEM((CH_D,), jnp.int32),
        pltpu.VMEM((CH_D,), jnp.int32),
        pltpu.VMEM((CH_D,), jnp.int32),
        pltpu.VMEM((CH_D, DO), jnp.float32),
        pltpu.VMEM((CH_D, DO), jnp.float32),
        pltpu.VMEM((1000, DO), jnp.float32),
        pltpu.SemaphoreType.DMA,
        pltpu.SemaphoreType.DMA,
        pltpu.SemaphoreType.DMA,
        pltpu.SemaphoreType.DMA,
        pltpu.SemaphoreType.DMA,
        pltpu.SemaphoreType.DMA,
        pltpu.VMEM_SHARED((N + 128, DO), jnp.float32),
    ],
)
def _s2_kernel(ts_hbm, src_hbm, dst_hbm, zeros_hbm, out_hbm,
               src0, src1, dst0, dst1, sidx0, sidx1, sdst0, sdst1,
               rows0, rows1, zbuf,
               isem0, isem1, gsem0, gsem1, ssem0, ssem1, acc):
    c = lax.axis_index("c")
    s = lax.axis_index("s")
    w = s * NC + c
    srcb = (src0, src1)
    dstb = (dst0, dst1)
    sidxb = (sidx0, sidx1)
    sdstb = (sdst0, sdst1)
    rowsb = (rows0, rows1)
    isems = (isem0, isem1)
    gsems = (gsem0, gsem1)
    ssems = (ssem0, ssem1)
    pltpu.sync_copy(zeros_hbm, zbuf)

    @pl.when(s < 10)
    def _():
        pltpu.sync_copy(zbuf, acc.at[pl.ds(s * 1000, 1000)])

    plsc.subcore_barrier()

    def off_of(k):
        return (w + k * NW) * CH_D

    def load_idx(k, b):
        pltpu.async_copy(src_hbm.at[pl.ds(off_of(k), CH_D)], srcb[b], isems[b])
        pltpu.async_copy(dst_hbm.at[pl.ds(off_of(k), CH_D)], dstb[b], isems[b])

    def wait_idx(b):
        pltpu.make_async_copy(src_hbm.at[pl.ds(0, CH_D)], srcb[b],
                              isems[b]).wait()
        pltpu.make_async_copy(dst_hbm.at[pl.ds(0, CH_D)], dstb[b],
                              isems[b]).wait()

    load_idx(0, 0)
    wait_idx(0)
    _vcopy(src0, sidx0, CH_D)
    pltpu.async_copy(ts_hbm.at[sidx0], rows0, gsem0)
    load_idx(1, 1)

    def iter_k(k, b):
        nb = 1 - b
        pltpu.make_async_copy(ts_hbm.at[sidxb[b]], rowsb[b], gsems[b]).wait()
        _vcopy(dstb[b], sdstb[b], CH_D)

        @pl.when(k + 1 < ND)
        def _():
            wait_idx(nb)

            @pl.when(k >= 1)
            def _():
                pltpu.make_async_copy(rowsb[nb], acc.at[sdstb[nb]],
                                      ssems[nb]).wait()

            _vcopy(srcb[nb], sidxb[nb], CH_D)
            pltpu.async_copy(ts_hbm.at[sidxb[nb]], rowsb[nb], gsems[nb])

        pltpu.async_copy(rowsb[b], acc.at[sdstb[b]], ssems[b], add=True)

        @pl.when(k + 2 < ND)
        def _():
            load_idx(k + 2, b)

    def pair(k2, carry):
        iter_k(2 * k2, 0)
        iter_k(2 * k2 + 1, 1)
        return carry

    lax.fori_loop(0, ND // 2, pair, 0)
    if ND % 2:
        iter_k(ND - 1, (ND - 1) % 2)
    pltpu.make_async_copy(rowsb[(ND - 2) % 2], acc.at[sdstb[(ND - 2) % 2]],
                          ssems[(ND - 2) % 2]).wait()
    pltpu.make_async_copy(rowsb[(ND - 1) % 2], acc.at[sdstb[(ND - 1) % 2]],
                          ssems[(ND - 1) % 2]).wait()
    plsc.subcore_barrier()

    @pl.when(s < 10)
    def _():
        pltpu.sync_copy(acc.at[pl.ds(s * 1000, 1000)], zbuf)
        pltpu.sync_copy(zbuf, out_hbm.at[pl.ds(c * N + s * 1000, 1000)])


def _scale_body(x_ref, degp_ref, xs_ref, dinv_ref):
    deg = degp_ref[0, :, :1] + degp_ref[1, :, :1] + 1.0
    dinv = lax.rsqrt(deg)
    dinv_ref[...] = dinv
    xb = x_ref[...] * dinv
    xs_ref[0] = xb[:, :HALF]
    xs_ref[1] = xb[:, HALF:]


def _tc_scale(x, degp):
    return pl.pallas_call(
        _scale_body,
        grid=(N // BK,),
        in_specs=[
            pl.BlockSpec((BK, DI), lambda i: (i, 0)),
            pl.BlockSpec((2, BK, 8), lambda i: (0, i, 0)),
        ],
        out_specs=[
            pl.BlockSpec((2, BK, HALF), lambda i: (0, i, 0)),
            pl.BlockSpec((BK, 1), lambda i: (i, 0)),
        ],
        out_shape=[
            jax.ShapeDtypeStruct((2, N, HALF), jnp.float32),
            jax.ShapeDtypeStruct((N, 1), jnp.float32),
        ],
    )(x, degp)


def _mm_body(s1a, s1b, xsa, xsb, dinv, w1a, w1b, b1r, w2, ts_ref):
    dv = dinv[...]
    a = (s1a[...] + xsa[...]) * dv
    b = (s1b[...] + xsb[...]) * dv
    h = jnp.dot(a, w1a[...], preferred_element_type=jnp.float32)
    h = h + jnp.dot(b, w1b[...], preferred_element_type=jnp.float32)
    h = jnp.maximum(h + b1r[...], 0.0)
    t = jnp.dot(h, w2[...], preferred_element_type=jnp.float32)
    ts_ref[...] = t * dv


def _tc_mm(s1a, s1b, xsa, xsb, dinv, W1a, W1b, b1r, W2):
    full = lambda shape: pl.BlockSpec(shape, lambda i: tuple(0 for _ in shape))
    return pl.pallas_call(
        _mm_body,
        grid=(N // BK,),
        in_specs=[
            pl.BlockSpec((BK, HALF), lambda i: (i, 0)),
            pl.BlockSpec((BK, HALF), lambda i: (i, 0)),
            pl.BlockSpec((BK, HALF), lambda i: (i, 0)),
            pl.BlockSpec((BK, HALF), lambda i: (i, 0)),
            pl.BlockSpec((BK, 1), lambda i: (i, 0)),
            full((HALF, DH)),
            full((HALF, DH)),
            full((1, DH)),
            full((DH, DO)),
        ],
        out_specs=pl.BlockSpec((BK, DO), lambda i: (i, 0)),
        out_shape=jax.ShapeDtypeStruct((N, DO), jnp.float32),
    )(s1a, s1b, xsa, xsb, dinv, W1a, W1b, b1r, W2)


def _fin_body(s2_ref, ts_ref, dinv, b2r, out_ref):
    acc = s2_ref[0] + s2_ref[1] + ts_ref[...]
    out_ref[...] = acc * dinv[...] + b2r[...]


def _tc_fin(s2, ts, dinv, b2r):
    return pl.pallas_call(
        _fin_body,
        grid=(N // BK,),
        in_specs=[
            pl.BlockSpec((2, BK, DO), lambda i: (0, i, 0)),
            pl.BlockSpec((BK, DO), lambda i: (i, 0)),
            pl.BlockSpec((BK, 1), lambda i: (i, 0)),
            pl.BlockSpec((1, DO), lambda i: (0, 0)),
        ],
        out_specs=pl.BlockSpec((BK, DO), lambda i: (i, 0)),
        out_shape=jax.ShapeDtypeStruct((N, DO), jnp.float32),
    )(s2, ts, dinv, b2r)


def kernel(x, edge_index, W1, b1, W2, b2):
    x = x.astype(jnp.float32)
    pad = NROW * CH - E
    src = jnp.concatenate(
        [edge_index[0].astype(jnp.int32), jnp.zeros((pad,), jnp.int32)])
    dst = jnp.concatenate(
        [edge_index[1].astype(jnp.int32),
         N + (jnp.arange(pad, dtype=jnp.int32) % 128)])

    ones_c = jnp.ones((CH_D, 8), jnp.float32)
    zeros_1 = jnp.zeros((1000, 8), jnp.float32)
    zeros_h = jnp.zeros((125, HALF), jnp.float32)
    zeros_o = jnp.zeros((1000, DO), jnp.float32)

    degp = _deg_kernel(dst, ones_c, zeros_1)
    xs, dinv = _tc_scale(x, degp.reshape(NC, N, 8))

    s1 = _s1_kernel(xs.reshape(NC * N, HALF), src, dst, zeros_h)
    s1 = s1.reshape(NC, N, HALF)

    ts = _tc_mm(s1[0], s1[1], xs[0], xs[1], dinv,
                W1[:HALF], W1[HALF:], b1.reshape(1, DH), W2)

    s2 = _s2_kernel(ts, src, dst, zeros_o)
    return _tc_fin(s2.reshape(NC, N, DO), ts, dinv, b2.reshape(1, DO))
